# baseline (speedup 1.0000x reference)
import functools
import jax
import jax.numpy as jnp
from jax import lax
from jax.experimental import pallas as pl
from jax.experimental.pallas import tpu as pltpu
from jax.experimental.pallas import tpu_sc as plsc

_L = 16
_C = 64
_NBUF = 2


def _sc_body(tot, D, nw, x_hbm, pos_hbm, tab_hbm, out_hbm,
             tab_v, bufs, idxs, in_sems, out_sems, idx_sems):
    rows_per_w = tot // nw
    nchunks = rows_per_w // _C
    wid = lax.axis_index("s") * 2 + lax.axis_index("c")
    row0 = wid * rows_per_w

    pltpu.sync_copy(tab_hbm, tab_v)

    def start_in(g, slot):
        base = (row0 + g * _C) * D
        pltpu.make_async_copy(
            x_hbm.at[pl.ds(base, _C * D)], bufs.at[slot], in_sems.at[slot]
        ).start()
        pltpu.make_async_copy(
            pos_hbm.at[pl.ds(row0 + g * _C, _C)], idxs.at[slot],
            idx_sems.at[slot],
        ).start()

    def wait_in(slot):
        pltpu.make_async_copy(
            x_hbm.at[pl.ds(0, _C * D)], bufs.at[slot], in_sems.at[slot]
        ).wait()
        pltpu.make_async_copy(
            pos_hbm.at[pl.ds(0, _C)], idxs.at[slot], idx_sems.at[slot]
        ).wait()

    def start_out(g, slot):
        base = (row0 + g * _C) * D
        pltpu.make_async_copy(
            bufs.at[slot], out_hbm.at[pl.ds(base, _C * D)], out_sems.at[slot]
        ).start()

    def wait_out(slot):
        pltpu.make_async_copy(
            bufs.at[slot], out_hbm.at[pl.ds(0, _C * D)], out_sems.at[slot]
        ).wait()

    for g in range(_NBUF - 1):
        start_in(g, g)

    def chunk_step(g, _):
        slot = lax.rem(g, _NBUF)
        nslot = lax.rem(g + _NBUF - 1, _NBUF)

        @pl.when(g + _NBUF - 1 < nchunks)
        def _():
            @pl.when(g >= 1)
            def _():
                wait_out(nslot)
            start_in(g + _NBUF - 1, nslot)

        wait_in(slot)

        @plsc.parallel_loop(0, _C // _L, 1, unroll=2)
        def group_step(grp):
            posv = idxs[slot, pl.ds(grp * _L, _L)]
            for rr in range(_L):
                tbase = posv[rr] * D
                xbase = (grp * _L + rr) * D
                for c in range(D // _L):
                    off = c * _L
                    x = bufs[slot, pl.ds(xbase + off, _L)]
                    t = tab_v[pl.ds(tbase + off, _L)]
                    bufs[slot, pl.ds(xbase + off, _L)] = x + t

        start_out(g, slot)
        return 0

    lax.fori_loop(0, nchunks, chunk_step, 0, unroll=False)
    for s in range(_NBUF):
        wait_out(s)


def kernel(inputs, inputs_positions, position_emb):
    B, N, D = inputs.shape
    tot = B * N
    info = plsc.get_sparse_core_info()
    nw = info.num_cores * info.num_subcores

    x = inputs.reshape(tot * D)
    pos = inputs_positions.reshape(tot).astype(jnp.int32)
    table = jnp.squeeze(position_emb, axis=0).reshape(-1)

    mesh = plsc.VectorSubcoreMesh(core_axis_name="c", subcore_axis_name="s")
    out = pl.kernel(
        functools.partial(_sc_body, tot, D, nw),
        out_type=jax.ShapeDtypeStruct((tot * D,), jnp.float32),
        mesh=mesh,
        scratch_types=[
            pltpu.VMEM((table.shape[0],), jnp.float32),
            pltpu.VMEM((_NBUF, _C * D), jnp.float32),
            pltpu.VMEM((_NBUF, _C), jnp.int32),
            pltpu.SemaphoreType.DMA((_NBUF,)),
            pltpu.SemaphoreType.DMA((_NBUF,)),
            pltpu.SemaphoreType.DMA((_NBUF,)),
        ],
    )(x, pos, table)
    return out.reshape(B, N, D)


# speedup vs baseline: 8.8721x; 8.8721x over previous
import jax
import jax.numpy as jnp
from jax.experimental import pallas as pl
from jax.experimental.pallas import tpu as pltpu

_BLOCK = 8192
_TPAD = 128


def _body(pos_ref, x_ref, tab_ref, o_ref):
    idx = pos_ref[0, 0, :]
    cols = jax.lax.broadcasted_iota(jnp.int32, (1, _TPAD), 1)
    onehot = (idx[:, None] == cols).astype(jnp.float32)
    g = jax.lax.dot_general(
        onehot, tab_ref[...], (((1,), (0,)), ((), ())),
        preferred_element_type=jnp.float32)
    o_ref[...] = x_ref[...] + g


def kernel(inputs, inputs_positions, position_emb):
    B, N, D = inputs.shape
    tot = B * N
    nb = tot // _BLOCK
    x = inputs.reshape(tot, D)
    pos = inputs_positions.reshape(nb, 1, _BLOCK).astype(jnp.int32)
    table = jnp.squeeze(position_emb, axis=0)
    table = jnp.pad(table, ((0, _TPAD - table.shape[0]), (0, 0)))

    out = pl.pallas_call(
        _body,
        grid=(nb,),
        in_specs=[
            pl.BlockSpec((1, 1, _BLOCK), lambda i: (i, 0, 0)),
            pl.BlockSpec((_BLOCK, D), lambda i: (i, 0)),
            pl.BlockSpec((_TPAD, D), lambda i: (0, 0)),
        ],
        out_specs=pl.BlockSpec((_BLOCK, D), lambda i: (i, 0)),
        out_shape=jax.ShapeDtypeStruct((tot, D), jnp.float32),
    )(pos, x, table)
    return out.reshape(B, N, D)
